# Initial kernel scaffold; baseline (speedup 1.0000x reference)
import jax
import jax.numpy as jnp
from jax.experimental import pallas as pl


def kernel(x, edge_index, Wl1, Wr1, b1, Wl2, Wr2, b2):
    raise NotImplementedError("write your pallas kernel here")


# speedup vs baseline: 3.3111x; 3.3111x over previous
import functools

import jax
import jax.numpy as jnp
from jax import lax
from jax.experimental import pallas as pl
from jax.experimental.pallas import tpu as pltpu
from jax.experimental.pallas import tpu_sc as plsc

NC = 2
NS = 16
BLK = 128
BR = 512


def _fill2d(ref, nrows, ncols, value):
    vv = jnp.full((16,), value, jnp.float32)
    k = ncols // 16

    def body(i, carry):
        r = i // k
        cidx = i % k
        ref[r, pl.ds(cidx * 16, 16)] = vv
        return carry

    lax.fori_loop(0, nrows * k, body, 0)


def _make_segsum(npad, d, nb, with_deg):
    rows_per_tile = npad // NS
    zchunks = rows_per_tile // BLK
    CH = 8

    out_type = [jax.ShapeDtypeStruct((NC, npad, d), jnp.float32)]
    scratch = [
        pltpu.VMEM((CH, BLK), jnp.int32),
        pltpu.VMEM((CH, BLK), jnp.int32),
        pltpu.VMEM((BLK, d), jnp.float32),
        pltpu.VMEM_SHARED((npad, d), jnp.float32),
        pltpu.SemaphoreType.DMA,
    ]
    if with_deg:
        out_type.append(jax.ShapeDtypeStruct((NC, npad, d), jnp.float32))

    mesh = plsc.VectorSubcoreMesh(core_axis_name="c", subcore_axis_name="s")

    def body(*refs):
        if with_deg:
            (y, srcb, dstb, part, deg_out, src_v, dst_v, rows_v, acc,
             sem) = refs
        else:
            (y, srcb, dstb, part, src_v, dst_v, rows_v, acc, sem) = refs

        c = lax.axis_index("c")
        s = lax.axis_index("s")
        wid = c * NS + s
        base = s * rows_per_tile

        def zero_acc():
            _fill2d(rows_v, BLK, d, 0.0)
            for k in range(zchunks):
                pltpu.sync_copy(rows_v,
                                acc.at[pl.ds(base + k * BLK, BLK), :])

        zero_acc()
        plsc.subcore_barrier()

        def chunk(o, carry):
            pltpu.sync_copy(srcb.at[wid, pl.ds(o * CH, CH)], src_v)
            pltpu.sync_copy(dstb.at[wid, pl.ds(o * CH, CH)], dst_v)

            def step(j, carry2):
                pltpu.async_copy(y.at[src_v.at[j]], rows_v, sem).wait()
                pltpu.sync_copy(rows_v, acc.at[dst_v.at[j]], add=True)
                return carry2

            return lax.fori_loop(0, CH, step, carry)

        lax.fori_loop(0, nb // CH, chunk, 0)
        plsc.subcore_barrier()

        pltpu.sync_copy(acc.at[pl.ds(base, rows_per_tile), :],
                        part.at[c, pl.ds(base, rows_per_tile), :])

        if with_deg:
            plsc.subcore_barrier()
            zero_acc()
            _fill2d(rows_v, BLK, d, 1.0)
            plsc.subcore_barrier()

            def dchunk(o, carry):
                pltpu.sync_copy(dstb.at[wid, pl.ds(o * CH, CH)], dst_v)

                def dstep(j, carry2):
                    pltpu.sync_copy(rows_v, acc.at[dst_v.at[j]], add=True)
                    return carry2

                return lax.fori_loop(0, CH, dstep, carry)

            lax.fori_loop(0, nb // CH, dchunk, 0)
            plsc.subcore_barrier()
            pltpu.sync_copy(acc.at[pl.ds(base, rows_per_tile), :],
                            deg_out.at[c, pl.ds(base, rows_per_tile), :])

    return pl.kernel(body, out_type=out_type, mesh=mesh,
                     scratch_types=scratch)


def _lin_body(x_ref, wl_ref, wr_ref, b_ref, y_ref, z_ref):
    xb = x_ref[...]
    dn = (((1,), (1,)), ((), ()))
    y_ref[...] = lax.dot_general(xb, wl_ref[...], dn,
                                 preferred_element_type=jnp.float32)
    z_ref[...] = lax.dot_general(xb, wr_ref[...], dn,
                                 preferred_element_type=jnp.float32) + b_ref[...]


def _lin(xp, wl, wr, b, npad, d):
    return pl.pallas_call(
        _lin_body,
        grid=(npad // BR,),
        in_specs=[pl.BlockSpec((BR, d), lambda i: (i, 0)),
                  pl.BlockSpec((d, d), lambda i: (0, 0)),
                  pl.BlockSpec((d, d), lambda i: (0, 0)),
                  pl.BlockSpec((1, d), lambda i: (0, 0))],
        out_specs=[pl.BlockSpec((BR, d), lambda i: (i, 0)),
                   pl.BlockSpec((BR, d), lambda i: (i, 0))],
        out_shape=[jax.ShapeDtypeStruct((npad, d), jnp.float32)] * 2,
    )(xp, wl, wr, b.reshape(1, d))


def _mid_body(part_ref, deg_ref, z1_ref, wl_ref, wr_ref, b_ref,
              y2_ref, z2_ref):
    p = part_ref[...]
    dg = deg_ref[...]
    inv = 1.0 / jnp.maximum(dg[0, :, :1] + dg[1, :, :1], 1.0)
    h = jnp.maximum((p[0] + p[1]) * inv + z1_ref[...], 0.0)
    dn = (((1,), (1,)), ((), ()))
    y2_ref[...] = lax.dot_general(h, wl_ref[...], dn,
                                  preferred_element_type=jnp.float32)
    z2_ref[...] = lax.dot_general(h, wr_ref[...], dn,
                                  preferred_element_type=jnp.float32) + b_ref[...]


def _mid(part, degr, z1, wl, wr, b, npad, d):
    return pl.pallas_call(
        _mid_body,
        grid=(npad // BR,),
        in_specs=[pl.BlockSpec((NC, BR, d), lambda i: (0, i, 0)),
                  pl.BlockSpec((NC, BR, d), lambda i: (0, i, 0)),
                  pl.BlockSpec((BR, d), lambda i: (i, 0)),
                  pl.BlockSpec((d, d), lambda i: (0, 0)),
                  pl.BlockSpec((d, d), lambda i: (0, 0)),
                  pl.BlockSpec((1, d), lambda i: (0, 0))],
        out_specs=[pl.BlockSpec((BR, d), lambda i: (i, 0)),
                   pl.BlockSpec((BR, d), lambda i: (i, 0))],
        out_shape=[jax.ShapeDtypeStruct((npad, d), jnp.float32)] * 2,
    )(part, degr, z1, wl, wr, b.reshape(1, d))


def _fin_body(part_ref, deg_ref, z2_ref, o_ref):
    p = part_ref[...]
    dg = deg_ref[...]
    inv = 1.0 / jnp.maximum(dg[0, :, :1] + dg[1, :, :1], 1.0)
    o = (p[0] + p[1]) * inv + z2_ref[...]
    m = jnp.max(o, axis=1, keepdims=True)
    e = jnp.exp(o - m)
    lse = jnp.log(jnp.sum(e, axis=1, keepdims=True))
    o_ref[...] = o - m - lse


def _fin(part, degr, z2, npad, d):
    return pl.pallas_call(
        _fin_body,
        grid=(npad // BR,),
        in_specs=[pl.BlockSpec((NC, BR, d), lambda i: (0, i, 0)),
                  pl.BlockSpec((NC, BR, d), lambda i: (0, i, 0)),
                  pl.BlockSpec((BR, d), lambda i: (i, 0))],
        out_specs=pl.BlockSpec((BR, d), lambda i: (i, 0)),
        out_shape=jax.ShapeDtypeStruct((npad, d), jnp.float32),
    )(part, degr, z2)


def kernel(x, edge_index, Wl1, Wr1, b1, Wl2, Wr2, b2):
    n, d = x.shape
    e = edge_index.shape[1]
    npad = -(-max(n + 1, BR) // BR) * BR
    nb = -(-(-(-e // (NC * NS * BLK))) // 8) * 8
    ep = NC * NS * nb * BLK

    ei = edge_index.astype(jnp.int32)
    src = ei[0]
    dst = ei[1]
    pad = ep - e
    if pad:
        src = jnp.concatenate([src, jnp.zeros((pad,), jnp.int32)])
        dst = jnp.concatenate([dst, jnp.full((pad,), n, jnp.int32)])
    srcb = src.reshape(NC * NS, nb, BLK)
    dstb = dst.reshape(NC * NS, nb, BLK)
    xp = jnp.pad(x, ((0, npad - n), (0, 0)))

    y1, z1 = _lin(xp, Wl1, Wr1, b1, npad, d)
    part1, deg = _make_segsum(npad, d, nb, True)(y1, srcb, dstb)
    y2, z2 = _mid(part1, deg, z1, Wl2, Wr2, b2, npad, d)
    part2 = _make_segsum(npad, d, nb, False)(y2, srcb, dstb)[0]
    out = _fin(part2, deg, z2, npad, d)
    return out[:n]


# speedup vs baseline: 3.5921x; 1.0849x over previous
import functools

import jax
import jax.numpy as jnp
from jax import lax
from jax.experimental import pallas as pl
from jax.experimental.pallas import tpu as pltpu
from jax.experimental.pallas import tpu_sc as plsc

NC = 2
NS = 16
BLK = 128
BR = 512
CH = 8


def _fill2d(ref, nrows, ncols, value):
    vv = jnp.full((16,), value, jnp.float32)
    k = ncols // 16

    def body(i, carry):
        r = i // k
        cidx = i % k
        ref[r, pl.ds(cidx * 16, 16)] = vv
        return carry

    lax.fori_loop(0, nrows * k, body, 0)


def _make_segsum(npad, d, nb, with_deg):
    rows_per_tile = npad // NS
    zchunks = rows_per_tile // BLK
    hr = npad // 128
    hwt = hr // 8

    out_type = [jax.ShapeDtypeStruct((NC, npad, d), jnp.float32)]
    scratch = [
        pltpu.VMEM((CH, BLK), jnp.int32),
        pltpu.VMEM((CH, BLK), jnp.int32),
        pltpu.VMEM((BLK, d), jnp.float32),
        pltpu.VMEM((BLK, d), jnp.float32),
        pltpu.VMEM_SHARED((npad, d), jnp.float32),
        pltpu.SemaphoreType.DMA,
        pltpu.SemaphoreType.DMA,
    ]
    if with_deg:
        out_type.append(jax.ShapeDtypeStruct((NC, hr, 128), jnp.float32))
        scratch += [
            pltpu.VMEM((hr, 128), jnp.float32),
            pltpu.VMEM((hr,), jnp.int32),
            pltpu.VMEM_SHARED((hr, 128), jnp.float32),
        ]

    mesh = plsc.VectorSubcoreMesh(core_axis_name="c", subcore_axis_name="s")

    def body(*refs):
        if with_deg:
            (y, srcb, dstb, part, deg_out, src_v, dst_v, rows_a, rows_b,
             acc, sem_a, sem_b, hist, hrows, sh_deg) = refs
        else:
            (y, srcb, dstb, part, src_v, dst_v, rows_a, rows_b, acc,
             sem_a, sem_b) = refs

        c = lax.axis_index("c")
        s = lax.axis_index("s")
        wid = c * NS + s
        base = s * rows_per_tile

        _fill2d(rows_a, BLK, d, 0.0)
        for k in range(zchunks):
            pltpu.sync_copy(rows_a, acc.at[pl.ds(base + k * BLK, BLK), :])
        if with_deg:
            _fill2d(hist, hr, 128, 0.0)
            iv = lax.iota(jnp.int32, 16)
            for r in range(hr // 16):
                hrows[pl.ds(r * 16, 16)] = iv + (16 * r)

            @pl.when(s < hwt)
            def _():
                pltpu.sync_copy(hist.at[pl.ds(0, 8)],
                                sh_deg.at[pl.ds(s * 8, 8), :])
        plsc.subcore_barrier()

        ones16 = jnp.full((16,), 1.0, jnp.float32)

        def chunk(o, carry):
            pltpu.sync_copy(srcb.at[wid, pl.ds(o * CH, CH)], src_v)
            pltpu.sync_copy(dstb.at[wid, pl.ds(o * CH, CH)], dst_v)

            def pair(p, carry2):
                j0 = 2 * p
                j1 = 2 * p + 1
                h0 = pltpu.async_copy(y.at[src_v.at[j0]], rows_a, sem_a)
                h1 = pltpu.async_copy(y.at[src_v.at[j1]], rows_b, sem_b)
                if with_deg:
                    for j in (j0, j1):
                        for k in range(BLK // 16):
                            dv = dst_v[j, pl.ds(k * 16, 16)]
                            row = jnp.right_shift(dv, 7)
                            col = jnp.bitwise_and(dv, 127)
                            plsc.addupdate_scatter(hist, [row, col], ones16)
                h0.wait()
                pltpu.sync_copy(rows_a, acc.at[dst_v.at[j0]], add=True)
                h1.wait()
                pltpu.sync_copy(rows_b, acc.at[dst_v.at[j1]], add=True)
                return carry2

            return lax.fori_loop(0, CH // 2, pair, carry)

        lax.fori_loop(0, nb // CH, chunk, 0)
        plsc.subcore_barrier()

        pltpu.sync_copy(acc.at[pl.ds(base, rows_per_tile), :],
                        part.at[c, pl.ds(base, rows_per_tile), :])

        if with_deg:
            pltpu.sync_copy(hist, sh_deg.at[hrows], add=True)
            plsc.subcore_barrier()

            @pl.when(s < hwt)
            def _():
                pltpu.sync_copy(sh_deg.at[pl.ds(s * 8, 8), :],
                                deg_out.at[c, pl.ds(s * 8, 8), :])

    return pl.kernel(
        body, out_type=out_type, mesh=mesh, scratch_types=scratch,
        compiler_params=pltpu.CompilerParams(needs_layout_passes=False))


def _lin_body(x_ref, wl_ref, wr_ref, b_ref, y_ref, z_ref):
    xb = x_ref[...]
    dn = (((1,), (1,)), ((), ()))
    y_ref[...] = lax.dot_general(xb, wl_ref[...], dn,
                                 preferred_element_type=jnp.float32)
    z_ref[...] = lax.dot_general(xb, wr_ref[...], dn,
                                 preferred_element_type=jnp.float32) + b_ref[...]


def _lin(xp, wl, wr, b, npad, d):
    return pl.pallas_call(
        _lin_body,
        grid=(npad // BR,),
        in_specs=[pl.BlockSpec((BR, d), lambda i: (i, 0)),
                  pl.BlockSpec((d, d), lambda i: (0, 0)),
                  pl.BlockSpec((d, d), lambda i: (0, 0)),
                  pl.BlockSpec((1, d), lambda i: (0, 0))],
        out_specs=[pl.BlockSpec((BR, d), lambda i: (i, 0)),
                   pl.BlockSpec((BR, d), lambda i: (i, 0))],
        out_shape=[jax.ShapeDtypeStruct((npad, d), jnp.float32)] * 2,
    )(xp, wl, wr, b.reshape(1, d))


def _mid_body(part_ref, deg_ref, z1_ref, wl_ref, wr_ref, b_ref,
              y2_ref, z2_ref):
    p = part_ref[...]
    dg = deg_ref[...]
    inv = 1.0 / jnp.maximum(dg[0] + dg[1], 1.0)
    h = jnp.maximum((p[0] + p[1]) * inv + z1_ref[...], 0.0)
    dn = (((1,), (1,)), ((), ()))
    y2_ref[...] = lax.dot_general(h, wl_ref[...], dn,
                                  preferred_element_type=jnp.float32)
    z2_ref[...] = lax.dot_general(h, wr_ref[...], dn,
                                  preferred_element_type=jnp.float32) + b_ref[...]


def _mid(part, degr, z1, wl, wr, b, npad, d):
    return pl.pallas_call(
        _mid_body,
        grid=(npad // BR,),
        in_specs=[pl.BlockSpec((NC, BR, d), lambda i: (0, i, 0)),
                  pl.BlockSpec((NC, BR, 1), lambda i: (0, i, 0)),
                  pl.BlockSpec((BR, d), lambda i: (i, 0)),
                  pl.BlockSpec((d, d), lambda i: (0, 0)),
                  pl.BlockSpec((d, d), lambda i: (0, 0)),
                  pl.BlockSpec((1, d), lambda i: (0, 0))],
        out_specs=[pl.BlockSpec((BR, d), lambda i: (i, 0)),
                   pl.BlockSpec((BR, d), lambda i: (i, 0))],
        out_shape=[jax.ShapeDtypeStruct((npad, d), jnp.float32)] * 2,
    )(part, degr, z1, wl, wr, b.reshape(1, d))


def _fin_body(part_ref, deg_ref, z2_ref, o_ref):
    p = part_ref[...]
    dg = deg_ref[...]
    inv = 1.0 / jnp.maximum(dg[0] + dg[1], 1.0)
    o = (p[0] + p[1]) * inv + z2_ref[...]
    m = jnp.max(o, axis=1, keepdims=True)
    e = jnp.exp(o - m)
    lse = jnp.log(jnp.sum(e, axis=1, keepdims=True))
    o_ref[...] = o - m - lse


def _fin(part, degr, z2, npad, d):
    return pl.pallas_call(
        _fin_body,
        grid=(npad // BR,),
        in_specs=[pl.BlockSpec((NC, BR, d), lambda i: (0, i, 0)),
                  pl.BlockSpec((NC, BR, 1), lambda i: (0, i, 0)),
                  pl.BlockSpec((BR, d), lambda i: (i, 0))],
        out_specs=pl.BlockSpec((BR, d), lambda i: (i, 0)),
        out_shape=jax.ShapeDtypeStruct((npad, d), jnp.float32),
    )(part, degr, z2)


def kernel(x, edge_index, Wl1, Wr1, b1, Wl2, Wr2, b2):
    n, d = x.shape
    e = edge_index.shape[1]
    npad = -(-max(n + 1, BR) // BR) * BR
    nb = -(-(-(-e // (NC * NS * BLK))) // CH) * CH
    ep = NC * NS * nb * BLK

    ei = edge_index.astype(jnp.int32)
    src = ei[0]
    dst = ei[1]
    pad = ep - e
    if pad:
        src = jnp.concatenate([src, jnp.zeros((pad,), jnp.int32)])
        dst = jnp.concatenate([dst, jnp.full((pad,), n, jnp.int32)])
    srcb = src.reshape(NC * NS, nb, BLK)
    dstb = dst.reshape(NC * NS, nb, BLK)
    xp = jnp.pad(x, ((0, npad - n), (0, 0)))

    y1, z1 = _lin(xp, Wl1, Wr1, b1, npad, d)
    part1, deg = _make_segsum(npad, d, nb, True)(y1, srcb, dstb)
    degr = deg.reshape(NC, npad, 1)
    y2, z2 = _mid(part1, degr, z1, Wl2, Wr2, b2, npad, d)
    part2 = _make_segsum(npad, d, nb, False)(y2, srcb, dstb)[0]
    out = _fin(part2, degr, z2, npad, d)
    return out[:n]
